# baseline (speedup 1.0000x reference)
import functools

import jax
import jax.numpy as jnp
from jax import lax
from jax.experimental import pallas as pl
from jax.experimental.pallas import tpu as pltpu
from jax.experimental.pallas import tpu_sc as plsc

B = 16384
DIM = 32
OCC_VOCAB = 22
NBUCKETS = 1000
TAB_ROWS = NBUCKETS + 1
TAB_WORDS = TAB_ROWS * DIM
TAB_PAD = 32128
NB_PAD = 1024
OUT_D = 5 * DIM + 2
MEAN = 0.5
VAR = 1.0 / 12.0
INV_STD = 1.0 / (VAR + 1e-6) ** 0.5

_info = plsc.get_sparse_core_info()
NC, NS, L = _info.num_cores, _info.num_subcores, _info.num_lanes
NW = NC * NS
ROWS_PER_W = B // NW
CHUNK = 128
NCHUNK = ROWS_PER_W // CHUNK
NGROUP = CHUNK // 16
NSTEP = 10


def _gather_sc(uid_hbm, rat_hbm, ts_hbm, utab4_hbm, rtab_hbm, ttab_hbm,
               rb_hbm, tb_hbm, ug_hbm, rg_hbm, tg_hbm,
               rtab_v, ttab_v, rb_v, tb_v, uid4_v,
               uid_v0, uid_v1, rat_v0, rat_v1, ts_v0, ts_v1,
               u4_rows, ue_v0, ue_v1, re_v0, re_v1, te_v0, te_v1,
               sem_tab, sem_in, sem_u, sem_out):
    uid_v = (uid_v0, uid_v1)
    rat_v = (rat_v0, rat_v1)
    ts_v = (ts_v0, ts_v1)
    ue_v = (ue_v0, ue_v1)
    re_v = (re_v0, re_v1)
    te_v = (te_v0, te_v1)
    wid = lax.axis_index("s") * NC + lax.axis_index("c")

    dtab = [pltpu.async_copy(rtab_hbm, rtab_v, sem_tab),
            pltpu.async_copy(ttab_hbm, ttab_v, sem_tab),
            pltpu.async_copy(rb_hbm, rb_v, sem_tab),
            pltpu.async_copy(tb_hbm, tb_v, sem_tab)]

    iota16 = lax.iota(jnp.int32, 16)

    def stage(j):
        rows = pl.ds(wid * ROWS_PER_W + j * CHUNK, CHUNK)
        b = j % 2
        return [pltpu.async_copy(uid_hbm.at[rows], uid_v[b], sem_in),
                pltpu.async_copy(rat_hbm.at[rows], rat_v[b], sem_in),
                pltpu.async_copy(ts_hbm.at[rows], ts_v[b], sem_in)]

    d_in = stage(0)
    d_out = []
    for j in range(NCHUNK):
        b = j % 2
        base = wid * ROWS_PER_W + j * CHUNK
        for d in d_in:
            d.wait()
        if j + 1 < NCHUNK:
            d_in = stage(j + 1)

        def quarter(g, _):
            uid4_v[pl.ds(g * 16, 16)] = uid_v[b][pl.ds(g * 16, 16)] >> 2
            return 0

        lax.fori_loop(0, NGROUP, quarter, 0)
        du = pltpu.async_copy(utab4_hbm.at[uid4_v], u4_rows, sem_u)

        if j == 0:
            for d in dtab:
                d.wait()
        if j >= 2:
            for d in d_out[j - 2]:
                d.wait()
            d_out[j - 2] = []

        def bucketize(g, _):
            gs = pl.ds(g * 16, 16)
            vr = rat_v[b][gs]
            vt = ts_v[b][gs]

            def search(bucket_ref, v):
                lo = jnp.zeros((16,), jnp.int32)
                hi = jnp.full((16,), NBUCKETS, jnp.int32)

                def step(_, carry):
                    lo, hi = carry
                    mid = (lo + hi) >> 1
                    p = plsc.load_gather(bucket_ref, [mid]) < v
                    return jnp.where(p, mid + 1, lo), jnp.where(p, hi, mid)

                return lax.fori_loop(0, NSTEP, step, (lo, hi))[0]

            ridx = search(rb_v, vr) * DIM
            tidx = search(tb_v, vt) * DIM
            dst = g * (16 * DIM) + iota16 * DIM
            for c in range(DIM):
                rv = plsc.load_gather(rtab_v, [ridx + c])
                tv = plsc.load_gather(ttab_v, [tidx + c])
                plsc.store_scatter(re_v[b], [dst + c], rv)
                plsc.store_scatter(te_v[b], [dst + c], tv)
            return 0

        lax.fori_loop(0, NGROUP, bucketize, 0)

        outs = pl.ds(base * DIM, CHUNK * DIM)
        dcs = [pltpu.async_copy(re_v[b], rg_hbm.at[outs], sem_out),
               pltpu.async_copy(te_v[b], tg_hbm.at[outs], sem_out)]
        du.wait()

        def extract(g, _):
            uidg = uid_v[b][pl.ds(g * 16, 16)]
            rows16 = g * 16 + iota16
            sub = (uidg & 3) * DIM
            dst = g * (16 * DIM) + iota16 * DIM
            for c in range(DIM):
                uv = plsc.load_gather(u4_rows, [rows16, sub + c])
                plsc.store_scatter(ue_v[b], [dst + c], uv)
            return 0

        lax.fori_loop(0, NGROUP, extract, 0)
        dcs.append(pltpu.async_copy(ue_v[b], ug_hbm.at[outs], sem_out))
        d_out.append(dcs)

    for dcs in d_out:
        for d in dcs:
            d.wait()


def _concat_tc(u_ref, r_ref, t_ref, gid_ref, oid_ref, rat_ref, ts_ref,
               gtab_ref, otab_ref, out_ref):
    br = u_ref.shape[0]
    g_oh = (gid_ref[...] == lax.broadcasted_iota(jnp.int32, (br, 2), 1)
            ).astype(jnp.float32)
    g_rows = jnp.dot(g_oh, gtab_ref[...], preferred_element_type=jnp.float32,
                     precision=lax.Precision.HIGHEST)
    o_oh = (oid_ref[...] == lax.broadcasted_iota(jnp.int32, (br, OCC_VOCAB), 1)
            ).astype(jnp.float32)
    o_rows = jnp.dot(o_oh, otab_ref[...], preferred_element_type=jnp.float32,
                     precision=lax.Precision.HIGHEST)
    nr = (rat_ref[...] - MEAN) * INV_STD
    nt = (ts_ref[...] - MEAN) * INV_STD
    out_ref[...] = jnp.concatenate(
        [u_ref[...], g_rows, o_rows, r_ref[...], nr, t_ref[...], nt], axis=1)


@jax.jit
def kernel(user_id, user_gender, user_occupation_label, user_rating, timestamp,
           user_table, gender_table, occupation_table, rating_table, timestamp_table,
           rating_buckets, timestamp_buckets):
    user_id = user_id.astype(jnp.int32)
    user_gender = user_gender.astype(jnp.int32)
    user_occupation_label = user_occupation_label.astype(jnp.int32)

    sc_gather = functools.partial(
        pl.kernel,
        out_type=[jax.ShapeDtypeStruct((B * DIM,), jnp.float32)] * 3,
        mesh=plsc.VectorSubcoreMesh(core_axis_name="c", subcore_axis_name="s"),
        compiler_params=pltpu.CompilerParams(needs_layout_passes=False),
        scratch_types=[
            pltpu.VMEM((TAB_PAD,), jnp.float32),
            pltpu.VMEM((TAB_PAD,), jnp.float32),
            pltpu.VMEM((NB_PAD,), jnp.float32),
            pltpu.VMEM((NB_PAD,), jnp.float32),
            pltpu.VMEM((CHUNK,), jnp.int32),
            pltpu.VMEM((CHUNK,), jnp.int32),
            pltpu.VMEM((CHUNK,), jnp.int32),
            pltpu.VMEM((CHUNK,), jnp.float32),
            pltpu.VMEM((CHUNK,), jnp.float32),
            pltpu.VMEM((CHUNK,), jnp.float32),
            pltpu.VMEM((CHUNK,), jnp.float32),
            pltpu.VMEM((CHUNK, 4 * DIM), jnp.float32),
            pltpu.VMEM((CHUNK * DIM,), jnp.float32),
            pltpu.VMEM((CHUNK * DIM,), jnp.float32),
            pltpu.VMEM((CHUNK * DIM,), jnp.float32),
            pltpu.VMEM((CHUNK * DIM,), jnp.float32),
            pltpu.VMEM((CHUNK * DIM,), jnp.float32),
            pltpu.VMEM((CHUNK * DIM,), jnp.float32),
            pltpu.SemaphoreType.DMA,
            pltpu.SemaphoreType.DMA,
            pltpu.SemaphoreType.DMA,
            pltpu.SemaphoreType.DMA,
        ],
    )(_gather_sc)
    ug, rg, tg = sc_gather(
        user_id, user_rating, timestamp,
        user_table.reshape(-1, 4 * DIM),
        jnp.pad(rating_table.reshape(TAB_WORDS), (0, TAB_PAD - TAB_WORDS)),
        jnp.pad(timestamp_table.reshape(TAB_WORDS), (0, TAB_PAD - TAB_WORDS)),
        jnp.pad(rating_buckets, (0, NB_PAD - NBUCKETS), constant_values=jnp.inf),
        jnp.pad(timestamp_buckets, (0, NB_PAD - NBUCKETS), constant_values=jnp.inf))

    br = 1024
    out = pl.pallas_call(
        _concat_tc,
        out_shape=jax.ShapeDtypeStruct((B, OUT_D), jnp.float32),
        grid=(B // br,),
        in_specs=[
            pl.BlockSpec((br, DIM), lambda i: (i, 0)),
            pl.BlockSpec((br, DIM), lambda i: (i, 0)),
            pl.BlockSpec((br, DIM), lambda i: (i, 0)),
            pl.BlockSpec((br, 1), lambda i: (i, 0)),
            pl.BlockSpec((br, 1), lambda i: (i, 0)),
            pl.BlockSpec((br, 1), lambda i: (i, 0)),
            pl.BlockSpec((br, 1), lambda i: (i, 0)),
            pl.BlockSpec((2, DIM), lambda i: (0, 0)),
            pl.BlockSpec((OCC_VOCAB, DIM), lambda i: (0, 0)),
        ],
        out_specs=pl.BlockSpec((br, OUT_D), lambda i: (i, 0)),
    )(ug.reshape(B, DIM), rg.reshape(B, DIM), tg.reshape(B, DIM),
      user_gender.reshape(B, 1), user_occupation_label.reshape(B, 1),
      user_rating.reshape(B, 1), timestamp.reshape(B, 1),
      gender_table, occupation_table)
    return out


# speedup vs baseline: 1.2321x; 1.2321x over previous
import functools

import jax
import jax.numpy as jnp
from jax import lax
from jax.experimental import pallas as pl
from jax.experimental.pallas import tpu as pltpu
from jax.experimental.pallas import tpu_sc as plsc

B = 16384
DIM = 32
OCC_VOCAB = 22
NBUCKETS = 1000
TAB_ROWS = NBUCKETS + 1
TAB_WORDS = TAB_ROWS * DIM
TAB_PAD = 32128
NB_PAD = 1024
OUT_D = 5 * DIM + 2
MEAN = 0.5
VAR = 1.0 / 12.0
INV_STD = 1.0 / (VAR + 1e-6) ** 0.5

_info = plsc.get_sparse_core_info()
NC, NS, L = _info.num_cores, _info.num_subcores, _info.num_lanes
NW = NC * NS
ROWS_PER_W = B // NW
CHUNK = 128
NCHUNK = ROWS_PER_W // CHUNK
NGROUP = CHUNK // 16
NSTEP = 10


def _gather_sc(uid_hbm, rat_hbm, ts_hbm, utab4_hbm, rtab_hbm, ttab_hbm,
               rb_hbm, tb_hbm, ug_hbm, rg_hbm, tg_hbm,
               rtab_v, ttab_v, rb_v, tb_v, uid4_v, rid_v, tid_v,
               uid_v0, uid_v1, rat_v0, rat_v1, ts_v0, ts_v1,
               u4_rows, ue_v0, ue_v1, re_v0, re_v1, te_v0, te_v1,
               sem_tab, sem_in, sem_u, sem_out):
    uid_v = (uid_v0, uid_v1)
    rat_v = (rat_v0, rat_v1)
    ts_v = (ts_v0, ts_v1)
    ue_v = (ue_v0, ue_v1)
    re_v = (re_v0, re_v1)
    te_v = (te_v0, te_v1)
    wid = lax.axis_index("s") * NC + lax.axis_index("c")

    dtab = [pltpu.async_copy(rtab_hbm, rtab_v, sem_tab),
            pltpu.async_copy(ttab_hbm, ttab_v, sem_tab),
            pltpu.async_copy(rb_hbm, rb_v, sem_tab),
            pltpu.async_copy(tb_hbm, tb_v, sem_tab)]

    iota16 = lax.iota(jnp.int32, 16)

    def stage(j):
        rows = pl.ds(wid * ROWS_PER_W + j * CHUNK, CHUNK)
        b = j % 2
        return [pltpu.async_copy(uid_hbm.at[rows], uid_v[b], sem_in),
                pltpu.async_copy(rat_hbm.at[rows], rat_v[b], sem_in),
                pltpu.async_copy(ts_hbm.at[rows], ts_v[b], sem_in)]

    d_in = stage(0)
    d_out = []
    for j in range(NCHUNK):
        b = j % 2
        base = wid * ROWS_PER_W + j * CHUNK
        for d in d_in:
            d.wait()
        if j + 1 < NCHUNK:
            d_in = stage(j + 1)

        def quarter(g, _):
            uid4_v[pl.ds(g * 16, 16)] = uid_v[b][pl.ds(g * 16, 16)] >> 2
            return 0

        lax.fori_loop(0, NGROUP, quarter, 0)
        du = pltpu.async_copy(utab4_hbm.at[uid4_v], u4_rows, sem_u)

        if j == 0:
            for d in dtab:
                d.wait()
        if j >= 2:
            for d in d_out[j - 2]:
                d.wait()
            d_out[j - 2] = []

        def bucketize(g, _):
            gs = pl.ds(g * 16, 16)
            vr = rat_v[b][gs]
            vt = ts_v[b][gs]

            def search(bucket_ref, v):
                lo = jnp.zeros((16,), jnp.int32)
                hi = jnp.full((16,), NBUCKETS, jnp.int32)

                def step(_, carry):
                    lo, hi = carry
                    mid = (lo + hi) >> 1
                    p = plsc.load_gather(bucket_ref, [mid]) < v
                    return jnp.where(p, mid + 1, lo), jnp.where(p, hi, mid)

                return lax.fori_loop(0, NSTEP, step, (lo, hi))[0]

            rid_v[gs] = search(rb_v, vr) * DIM
            tid_v[gs] = search(tb_v, vt) * DIM
            return 0

        lax.fori_loop(0, NGROUP, bucketize, 0)

        def lookup(g, _):
            rvec = rid_v[pl.ds(g * 16, 16)]
            tvec = tid_v[pl.ds(g * 16, 16)]
            for k in range(16):
                i = g * 16 + k
                ro = rvec[k]
                to = tvec[k]
                d = pl.ds(i * DIM, 16)
                d2 = pl.ds(i * DIM + 16, 16)
                re_v[b][d] = rtab_v[pl.ds(ro, 16)]
                re_v[b][d2] = rtab_v[pl.ds(ro + 16, 16)]
                te_v[b][d] = ttab_v[pl.ds(to, 16)]
                te_v[b][d2] = ttab_v[pl.ds(to + 16, 16)]
            return 0

        lax.fori_loop(0, NGROUP, lookup, 0)

        outs = pl.ds(base * DIM, CHUNK * DIM)
        dcs = [pltpu.async_copy(re_v[b], rg_hbm.at[outs], sem_out),
               pltpu.async_copy(te_v[b], tg_hbm.at[outs], sem_out)]
        du.wait()

        def extract(g, _):
            offv = (uid_v[b][pl.ds(g * 16, 16)] & 3) * DIM
            for k in range(16):
                i = g * 16 + k
                off = offv[k]
                ue_v[b][pl.ds(i * DIM, 16)] = u4_rows[i, pl.ds(off, 16)]
                ue_v[b][pl.ds(i * DIM + 16, 16)] = u4_rows[i, pl.ds(off + 16, 16)]
            return 0

        lax.fori_loop(0, NGROUP, extract, 0)
        dcs.append(pltpu.async_copy(ue_v[b], ug_hbm.at[outs], sem_out))
        d_out.append(dcs)

    for dcs in d_out:
        for d in dcs:
            d.wait()


def _concat_tc(u_ref, r_ref, t_ref, gid_ref, oid_ref, rat_ref, ts_ref,
               gtab_ref, otab_ref, out_ref):
    br = u_ref.shape[0]
    g_oh = (gid_ref[...] == lax.broadcasted_iota(jnp.int32, (br, 2), 1)
            ).astype(jnp.float32)
    g_rows = jnp.dot(g_oh, gtab_ref[...], preferred_element_type=jnp.float32,
                     precision=lax.Precision.HIGHEST)
    o_oh = (oid_ref[...] == lax.broadcasted_iota(jnp.int32, (br, OCC_VOCAB), 1)
            ).astype(jnp.float32)
    o_rows = jnp.dot(o_oh, otab_ref[...], preferred_element_type=jnp.float32,
                     precision=lax.Precision.HIGHEST)
    nr = (rat_ref[...] - MEAN) * INV_STD
    nt = (ts_ref[...] - MEAN) * INV_STD
    out_ref[...] = jnp.concatenate(
        [u_ref[...], g_rows, o_rows, r_ref[...], nr, t_ref[...], nt], axis=1)


@jax.jit
def kernel(user_id, user_gender, user_occupation_label, user_rating, timestamp,
           user_table, gender_table, occupation_table, rating_table, timestamp_table,
           rating_buckets, timestamp_buckets):
    user_id = user_id.astype(jnp.int32)
    user_gender = user_gender.astype(jnp.int32)
    user_occupation_label = user_occupation_label.astype(jnp.int32)

    sc_gather = functools.partial(
        pl.kernel,
        out_type=[jax.ShapeDtypeStruct((B * DIM,), jnp.float32)] * 3,
        mesh=plsc.VectorSubcoreMesh(core_axis_name="c", subcore_axis_name="s"),
        compiler_params=pltpu.CompilerParams(needs_layout_passes=False),
        scratch_types=[
            pltpu.VMEM((TAB_PAD,), jnp.float32),
            pltpu.VMEM((TAB_PAD,), jnp.float32),
            pltpu.VMEM((NB_PAD,), jnp.float32),
            pltpu.VMEM((NB_PAD,), jnp.float32),
            pltpu.VMEM((CHUNK,), jnp.int32),
            pltpu.VMEM((CHUNK,), jnp.int32),
            pltpu.VMEM((CHUNK,), jnp.int32),
            pltpu.VMEM((CHUNK,), jnp.int32),
            pltpu.VMEM((CHUNK,), jnp.int32),
            pltpu.VMEM((CHUNK,), jnp.float32),
            pltpu.VMEM((CHUNK,), jnp.float32),
            pltpu.VMEM((CHUNK,), jnp.float32),
            pltpu.VMEM((CHUNK,), jnp.float32),
            pltpu.VMEM((CHUNK, 4 * DIM), jnp.float32),
            pltpu.VMEM((CHUNK * DIM,), jnp.float32),
            pltpu.VMEM((CHUNK * DIM,), jnp.float32),
            pltpu.VMEM((CHUNK * DIM,), jnp.float32),
            pltpu.VMEM((CHUNK * DIM,), jnp.float32),
            pltpu.VMEM((CHUNK * DIM,), jnp.float32),
            pltpu.VMEM((CHUNK * DIM,), jnp.float32),
            pltpu.SemaphoreType.DMA,
            pltpu.SemaphoreType.DMA,
            pltpu.SemaphoreType.DMA,
            pltpu.SemaphoreType.DMA,
        ],
    )(_gather_sc)
    ug, rg, tg = sc_gather(
        user_id, user_rating, timestamp,
        user_table.reshape(-1, 4 * DIM),
        jnp.pad(rating_table.reshape(TAB_WORDS), (0, TAB_PAD - TAB_WORDS)),
        jnp.pad(timestamp_table.reshape(TAB_WORDS), (0, TAB_PAD - TAB_WORDS)),
        jnp.pad(rating_buckets, (0, NB_PAD - NBUCKETS), constant_values=jnp.inf),
        jnp.pad(timestamp_buckets, (0, NB_PAD - NBUCKETS), constant_values=jnp.inf))

    br = 1024
    out = pl.pallas_call(
        _concat_tc,
        out_shape=jax.ShapeDtypeStruct((B, OUT_D), jnp.float32),
        grid=(B // br,),
        in_specs=[
            pl.BlockSpec((br, DIM), lambda i: (i, 0)),
            pl.BlockSpec((br, DIM), lambda i: (i, 0)),
            pl.BlockSpec((br, DIM), lambda i: (i, 0)),
            pl.BlockSpec((br, 1), lambda i: (i, 0)),
            pl.BlockSpec((br, 1), lambda i: (i, 0)),
            pl.BlockSpec((br, 1), lambda i: (i, 0)),
            pl.BlockSpec((br, 1), lambda i: (i, 0)),
            pl.BlockSpec((2, DIM), lambda i: (0, 0)),
            pl.BlockSpec((OCC_VOCAB, DIM), lambda i: (0, 0)),
        ],
        out_specs=pl.BlockSpec((br, OUT_D), lambda i: (i, 0)),
    )(ug.reshape(B, DIM), rg.reshape(B, DIM), tg.reshape(B, DIM),
      user_gender.reshape(B, 1), user_occupation_label.reshape(B, 1),
      user_rating.reshape(B, 1), timestamp.reshape(B, 1),
      gender_table, occupation_table)
    return out


# speedup vs baseline: 1.7010x; 1.3805x over previous
import functools

import jax
import jax.numpy as jnp
from jax import lax
from jax.experimental import pallas as pl
from jax.experimental.pallas import tpu as pltpu
from jax.experimental.pallas import tpu_sc as plsc

B = 16384
DIM = 32
OCC_VOCAB = 22
NBUCKETS = 1000
TAB_ROWS = NBUCKETS + 1
OUT_D = 5 * DIM + 2
MEAN = 0.5
VAR = 1.0 / 12.0
INV_STD = 1.0 / (VAR + 1e-6) ** 0.5

_info = plsc.get_sparse_core_info()
NC, NS, L = _info.num_cores, _info.num_subcores, _info.num_lanes
NW = NC * NS
ROWS_PER_W = B // NW
CHUNK = 128
NCHUNK = ROWS_PER_W // CHUNK
NGROUP = CHUNK // 16
NSTEP = 10

RT_OFF = 0
TT_OFF = RT_OFF + DIM * TAB_ROWS
GT_OFF = TT_OFF + DIM * TAB_ROWS
OT_OFF = GT_OFF + DIM * 2
RB_OFF = OT_OFF + DIM * OCC_VOCAB
TB_OFF = RB_OFF + NBUCKETS + 1
BLOB_RAW = TB_OFF + NBUCKETS + 1
BLOB = (BLOB_RAW + 127) // 128 * 128


def _encode_sc(uid_hbm, gid_hbm, oid_hbm, rat_hbm, ts_hbm, u4tab_hbm, blob_hbm,
               out_hbm,
               blob_v, uid4_v, rid_v, tid_v,
               uid_v0, uid_v1, gid_v0, gid_v1, oid_v0, oid_v1,
               rat_v0, rat_v1, ts_v0, ts_v1,
               u4_rows, tile_v,
               sem_tab, sem_in, sem_u, sem_out):
    uid_v = (uid_v0, uid_v1)
    gid_v = (gid_v0, gid_v1)
    oid_v = (oid_v0, oid_v1)
    rat_v = (rat_v0, rat_v1)
    ts_v = (ts_v0, ts_v1)
    wid = lax.axis_index("s") * NC + lax.axis_index("c")

    dtab = pltpu.async_copy(blob_hbm, blob_v, sem_tab)

    iota16 = lax.iota(jnp.int32, 16)

    def stage(j):
        rows = pl.ds(wid * ROWS_PER_W + j * CHUNK, CHUNK)
        b = j % 2
        return [pltpu.async_copy(uid_hbm.at[rows], uid_v[b], sem_in),
                pltpu.async_copy(gid_hbm.at[rows], gid_v[b], sem_in),
                pltpu.async_copy(oid_hbm.at[rows], oid_v[b], sem_in),
                pltpu.async_copy(rat_hbm.at[rows], rat_v[b], sem_in),
                pltpu.async_copy(ts_hbm.at[rows], ts_v[b], sem_in)]

    d_in = stage(0)
    d_out = None
    for j in range(NCHUNK):
        b = j % 2
        base = wid * ROWS_PER_W + j * CHUNK
        for d in d_in:
            d.wait()
        if j + 1 < NCHUNK:
            d_in = stage(j + 1)

        def quarter(g, _):
            uid4_v[pl.ds(g * 16, 16)] = uid_v[b][pl.ds(g * 16, 16)] >> 2
            return 0

        lax.fori_loop(0, NGROUP, quarter, 0)
        du = pltpu.async_copy(u4tab_hbm.at[uid4_v], u4_rows, sem_u)

        if j == 0:
            dtab.wait()

        def bucketize(g, _):
            gs = pl.ds(g * 16, 16)
            vr = rat_v[b][gs]
            vt = ts_v[b][gs]

            def search(off, v):
                lo = jnp.zeros((16,), jnp.int32)
                hi = jnp.full((16,), NBUCKETS, jnp.int32)

                def step(_, carry):
                    lo, hi = carry
                    mid = (lo + hi) >> 1
                    p = plsc.load_gather(blob_v, [off + mid]) < v
                    return jnp.where(p, mid + 1, lo), jnp.where(p, hi, mid)

                return lax.fori_loop(0, NSTEP, step, (lo, hi))[0]

            rid_v[gs] = search(RB_OFF, vr)
            tid_v[gs] = search(TB_OFF, vt)
            return 0

        lax.fori_loop(0, NGROUP, bucketize, 0)

        if d_out is not None:
            d_out.wait()
        du.wait()

        rt_base0 = RT_OFF + iota16 * TAB_ROWS
        rt_base1 = RT_OFF + (iota16 + 16) * TAB_ROWS
        tt_base0 = TT_OFF + iota16 * TAB_ROWS
        tt_base1 = TT_OFF + (iota16 + 16) * TAB_ROWS
        gt_base0 = GT_OFF + iota16 * 2
        gt_base1 = GT_OFF + (iota16 + 16) * 2
        ot_base0 = OT_OFF + iota16 * OCC_VOCAB
        ot_base1 = OT_OFF + (iota16 + 16) * OCC_VOCAB

        def assemble(g, _):
            gs = pl.ds(g * 16, 16)
            offv = (uid_v[b][gs] & 3) * DIM
            gv = gid_v[b][gs]
            ov = oid_v[b][gs]
            rv = rid_v[gs]
            tv = tid_v[gs]
            for k in range(16):
                i = g * 16 + k
                uo = offv[k]
                gk = gv[k]
                ok = ov[k]
                rk = rv[k]
                tk = tv[k]
                tile_v[i, pl.ds(0, 16)] = u4_rows[i, pl.ds(uo, 16)]
                tile_v[i, pl.ds(16, 16)] = u4_rows[i, pl.ds(uo + 16, 16)]
                tile_v[i, pl.ds(32, 16)] = plsc.load_gather(blob_v, [gt_base0 + gk])
                tile_v[i, pl.ds(48, 16)] = plsc.load_gather(blob_v, [gt_base1 + gk])
                tile_v[i, pl.ds(64, 16)] = plsc.load_gather(blob_v, [ot_base0 + ok])
                tile_v[i, pl.ds(80, 16)] = plsc.load_gather(blob_v, [ot_base1 + ok])
                tile_v[i, pl.ds(96, 16)] = plsc.load_gather(blob_v, [rt_base0 + rk])
                tile_v[i, pl.ds(112, 16)] = plsc.load_gather(blob_v, [rt_base1 + rk])
                tile_v[i, pl.ds(129, 16)] = plsc.load_gather(blob_v, [tt_base0 + tk])
                tile_v[i, pl.ds(145, 16)] = plsc.load_gather(blob_v, [tt_base1 + tk])
            rows16 = g * 16 + iota16
            nr = (rat_v[b][gs] - MEAN) * INV_STD
            nt = (ts_v[b][gs] - MEAN) * INV_STD
            plsc.store_scatter(tile_v, [rows16, jnp.full((16,), 4 * DIM, jnp.int32)], nr)
            plsc.store_scatter(tile_v, [rows16, jnp.full((16,), 5 * DIM + 1, jnp.int32)], nt)
            return 0

        lax.fori_loop(0, NGROUP, assemble, 0)

        d_out = pltpu.async_copy(tile_v, out_hbm.at[pl.ds(base, CHUNK), :], sem_out)

    d_out.wait()


@jax.jit
def kernel(user_id, user_gender, user_occupation_label, user_rating, timestamp,
           user_table, gender_table, occupation_table, rating_table, timestamp_table,
           rating_buckets, timestamp_buckets):
    user_id = user_id.astype(jnp.int32)
    user_gender = user_gender.astype(jnp.int32)
    user_occupation_label = user_occupation_label.astype(jnp.int32)

    blob = jnp.concatenate([
        rating_table.T.reshape(-1), timestamp_table.T.reshape(-1),
        gender_table.T.reshape(-1), occupation_table.T.reshape(-1),
        rating_buckets, jnp.full((1,), jnp.inf, jnp.float32),
        timestamp_buckets, jnp.full((1,), jnp.inf, jnp.float32),
        jnp.zeros((BLOB - BLOB_RAW,), jnp.float32)])

    run = functools.partial(
        pl.kernel,
        out_type=jax.ShapeDtypeStruct((B, OUT_D), jnp.float32),
        mesh=plsc.VectorSubcoreMesh(core_axis_name="c", subcore_axis_name="s"),
        compiler_params=pltpu.CompilerParams(needs_layout_passes=False),
        scratch_types=[
            pltpu.VMEM((BLOB,), jnp.float32),
            pltpu.VMEM((CHUNK,), jnp.int32),
            pltpu.VMEM((CHUNK,), jnp.int32),
            pltpu.VMEM((CHUNK,), jnp.int32),
            pltpu.VMEM((CHUNK,), jnp.int32),
            pltpu.VMEM((CHUNK,), jnp.int32),
            pltpu.VMEM((CHUNK,), jnp.int32),
            pltpu.VMEM((CHUNK,), jnp.int32),
            pltpu.VMEM((CHUNK,), jnp.int32),
            pltpu.VMEM((CHUNK,), jnp.int32),
            pltpu.VMEM((CHUNK,), jnp.float32),
            pltpu.VMEM((CHUNK,), jnp.float32),
            pltpu.VMEM((CHUNK,), jnp.float32),
            pltpu.VMEM((CHUNK,), jnp.float32),
            pltpu.VMEM((CHUNK, 4 * DIM), jnp.float32),
            pltpu.VMEM((CHUNK, OUT_D), jnp.float32),
            pltpu.SemaphoreType.DMA,
            pltpu.SemaphoreType.DMA,
            pltpu.SemaphoreType.DMA,
            pltpu.SemaphoreType.DMA,
        ],
    )(_encode_sc)
    return run(user_id, user_gender, user_occupation_label, user_rating, timestamp,
               user_table.reshape(-1, 4 * DIM), blob)
